# baseline (speedup 1.0000x reference)
import functools

import jax
import jax.numpy as jnp
from jax import lax
from jax.experimental import pallas as pl
from jax.experimental.pallas import tpu as pltpu
from jax.experimental.pallas import tpu_sc as plsc

_NUM_CORES = 2
_NUM_SUBCORES = 16
_NUM_WORKERS = _NUM_CORES * _NUM_SUBCORES
_LANES = 16

_VOCAB = 1000000
_BATCH = 4096
_MAXLEN = 200
_EMBED = 64
_SEQ_PER_W = _BATCH // _NUM_WORKERS
_HALF = _MAXLEN // 2

_TB = 128
_NFULL = _VOCAB // _TB
_BLK_PER_W = -(-_NFULL // _NUM_WORKERS)


def _transpose_body(tokT_hbm, tail_hbm, out_hbm, staged, outbuf, sems):
    wid = lax.axis_index("s") * _NUM_CORES + lax.axis_index("c")

    iot = lax.iota(jnp.int32, _LANES)
    row_base = lax.shift_right_logical(iot, 1)
    col_par = lax.bitwise_and(iot, 1) * _EMBED

    def stage_start(b, k):
        off = pl.multiple_of(b * _TB, _TB)
        pltpu.async_copy(tokT_hbm.at[:, pl.ds(off, _TB)], staged.at[k], sems[k])

    def stage_wait(b, k):
        off = pl.multiple_of(b * _TB, _TB)
        pltpu.make_async_copy(
            tokT_hbm.at[:, pl.ds(off, _TB)], staged.at[k], sems[k]
        ).wait()

    def transpose(k):
        def erow(e, carry):
            col_idx = col_par + e

            def t8(t, carry2):
                v = plsc.load_gather(
                    staged.at[k],
                    [jnp.full((_LANES,), e, jnp.int32), iot + t * _LANES])
                row_idx = row_base + t * (_LANES // 2)
                plsc.store_scatter(outbuf.at[k], [row_idx, col_idx], v)
                return carry2

            return lax.fori_loop(0, _TB // _LANES, t8, carry)

        lax.fori_loop(0, _EMBED, erow, 0)

    def wb_start(b, k):
        off = pl.multiple_of(b * (_TB // 2), _TB // 2)
        pltpu.async_copy(outbuf.at[k], out_hbm.at[pl.ds(off, _TB // 2)],
                         sems[2 + k])

    def wb_wait(b, k):
        off = pl.multiple_of(b * (_TB // 2), _TB // 2)
        pltpu.make_async_copy(
            outbuf.at[k], out_hbm.at[pl.ds(off, _TB // 2)], sems[2 + k]
        ).wait()

    def blk(g):
        return wid + g * _NUM_WORKERS

    stage_start(blk(0), 0)

    def pair(g2, carry):
        for kk in range(2):
            g = g2 * 2 + kk
            b = blk(g)

            @pl.when(b < _NFULL)
            def _():
                @pl.when(blk(g + 1) < _NFULL)
                def _():
                    stage_start(blk(g + 1), 1 - kk)
                stage_wait(b, kk)

                @pl.when(g >= 2)
                def _():
                    wb_wait(blk(g - 2), kk)
                transpose(kk)
                wb_start(b, kk)
        return carry

    lax.fori_loop(0, (_BLK_PER_W + 1) // 2, pair, 0, unroll=1)

    g_max = lax.div(_NFULL - 1 - wid, _NUM_WORKERS)
    for kk in range(2):
        g_k = g_max - lax.rem(g_max - kk + 2, 2)

        @pl.when(g_k >= 0)
        def _():
            wb_wait(blk(g_k), kk)

    @pl.when(wid == 0)
    def _():
        pltpu.sync_copy(tail_hbm, staged.at[0])
        transpose(0)
        pltpu.sync_copy(outbuf.at[0],
                        out_hbm.at[pl.ds((_VOCAB - _TB) // 2, _TB // 2)])


def _emb_body(x_hbm, tok_hbm, pos_hbm, out_hbm, idx_v, pos_v, rows_v,
              g0, g1, g2, g3, w0, w1, w2, w3):
    gsems = (g0, g1, g2, g3)
    wsems = (w0, w1, w2, w3)
    wid = lax.axis_index("s") * _NUM_CORES + lax.axis_index("c")
    base_seq = wid * _SEQ_PER_W

    pltpu.sync_copy(x_hbm.at[pl.ds(base_seq, _SEQ_PER_W)], idx_v)
    pltpu.sync_copy(pos_hbm, pos_v)

    def start_gather(s, b):
        for j in range(2):
            pltpu.async_copy(
                tok_hbm.at[idx_v.at[s, j]],
                rows_v.at[b, pl.ds(j * _HALF, _HALF)],
                gsems[b],
            )

    def wait_gather(s, b):
        for j in range(2):
            pltpu.make_async_copy(
                tok_hbm.at[idx_v.at[s, j]],
                rows_v.at[b, pl.ds(j * _HALF, _HALF)],
                gsems[b],
            ).wait()

    def pos_add(b):
        def row(m, carry):
            for l in range(_EMBED // _LANES):
                p = pos_v[m, pl.ds(l * _LANES, _LANES)]
                plsc.addupdate(rows_v.at[b, m, pl.ds(l * _LANES, _LANES)], p)
            return carry

        lax.fori_loop(0, _MAXLEN, row, 0, unroll=2)

    def start_wb(s, b):
        pltpu.async_copy(rows_v.at[b], out_hbm.at[base_seq + s], wsems[b])

    def wait_wb(s, b):
        pltpu.make_async_copy(rows_v.at[b], out_hbm.at[base_seq + s],
                              wsems[b]).wait()

    _NBUF = 4
    _GDEPTH = 2
    for b in range(_GDEPTH):
        start_gather(b, b)

    def outer(g, carry):
        for b in range(_NBUF):
            s = g * _NBUF + b
            wait_gather(s, b)
            pos_add(b)
            start_wb(s, b)
            bn = (b + _GDEPTH) % _NBUF
            sn = s + _GDEPTH

            @pl.when(sn < _SEQ_PER_W)
            def _():
                @pl.when(sn >= _NBUF)
                def _():
                    wait_wb(sn - _NBUF, bn)
                start_gather(sn, bn)
        return carry

    lax.fori_loop(0, _SEQ_PER_W // _NBUF, outer, 0)

    for b in range(_NBUF):
        wait_wb(_SEQ_PER_W - _NBUF + b, b)


def _mesh():
    return plsc.VectorSubcoreMesh(
        core_axis_name="c", subcore_axis_name="s",
        num_cores=_NUM_CORES, num_subcores=_NUM_SUBCORES,
    )


def _transpose(tokT, tail):
    return pl.kernel(
        _transpose_body,
        out_type=jax.ShapeDtypeStruct((_VOCAB // 2, 2 * _EMBED), jnp.float32),
        mesh=_mesh(),
        compiler_params=pltpu.CompilerParams(
            use_tc_tiling_on_sc=True, needs_layout_passes=False),
        scratch_types=[
            pltpu.VMEM((2, _EMBED, _TB), jnp.float32),
            pltpu.VMEM((2, _TB // 2, 2 * _EMBED), jnp.float32),
            [pltpu.SemaphoreType.DMA] * 4,
        ],
    )(tokT, tail)


def _gather_add(x3, tok_rm, pos_table):
    return pl.kernel(
        _emb_body,
        out_type=jax.ShapeDtypeStruct((_BATCH, _MAXLEN, _EMBED), jnp.float32),
        mesh=_mesh(),
        compiler_params=pltpu.CompilerParams(use_tc_tiling_on_sc=False),
        scratch_types=[
            pltpu.VMEM((_SEQ_PER_W, 2, _HALF), jnp.int32),
            pltpu.VMEM((_MAXLEN, _EMBED), jnp.float32),
            pltpu.VMEM((4, _MAXLEN, _EMBED), jnp.float32),
        ] + [pltpu.SemaphoreType.DMA] * 8,
    )(x3, tok_rm, pos_table)


@jax.jit
def _run(x3, token_table, pos_table):
    tokT = token_table.T
    tail = token_table[_VOCAB - _TB:, :].T
    t128 = _transpose(tokT, tail)
    tok_rm = t128.reshape(_VOCAB, _EMBED)
    return _gather_add(x3, tok_rm, pos_table)


def kernel(x, token_table, pos_table):
    x3 = jnp.asarray(x, jnp.int32).reshape(_BATCH, 2, _HALF)
    return _run(x3, token_table, pos_table)
